# Initial kernel scaffold; baseline (speedup 1.0000x reference)
import jax
import jax.numpy as jnp
from jax.experimental import pallas as pl


def kernel(coords, edge_index, W_node, b_node, W_g0, b_g0, W_g1, b_g1, W_g2, b_g2, W_edge, b_edge):
    raise NotImplementedError("write your pallas kernel here")


# speedup vs baseline: 3.0747x; 3.0747x over previous
import functools

import jax
import jax.numpy as jnp
from jax import lax
from jax.experimental import pallas as pl
from jax.experimental.pallas import tpu as pltpu
from jax.experimental.pallas import tpu_sc as plsc

N_NODES = 50000
N_EDGES = 800000
D = 64
H = 32

NC = 2
NS = 16
CH = 128

N_PAD = 50176
ROWS_PER_TILE = N_PAD // NS
ZROWS = ROWS_PER_TILE // 4
E_PAD = 819200
CHUNKS_PER_TILE = E_PAD // (NS * CH)

N_CHUNKS_EDGE = N_EDGES // CH
N_WORKERS = NC * NS
EDGE_ITERS = (N_CHUNKS_EDGE + N_WORKERS - 1) // N_WORKERS

@functools.lru_cache(maxsize=None)
def _mesh():
    return plsc.VectorSubcoreMesh(
        core_axis_name="c", subcore_axis_name="s",
        num_cores=NC, num_subcores=NS)


def _sc_segsum_body(hL, hR, srcp, dstp, aggL, aggR,
                    acc, zv, sv, dv, rows, sem):
    c = lax.axis_index("c")
    s = lax.axis_index("s")

    @pl.loop(0, ZROWS)
    def _zero(r):
        zv[r, 0:16] = jnp.zeros((16,), jnp.float32)
        zv[r, 16:32] = jnp.zeros((16,), jnp.float32)

    for j in range(ROWS_PER_TILE // ZROWS):
        pltpu.sync_copy(zv, acc.at[pl.ds(s * ROWS_PER_TILE + j * ZROWS, ZROWS)])
    plsc.subcore_barrier()

    @pl.loop(0, CHUNKS_PER_TILE)
    def _chunk(i):
        base = s * (CHUNKS_PER_TILE * CH) + i * CH
        pltpu.sync_copy(srcp.at[pl.ds(base, CH)], sv)
        pltpu.sync_copy(dstp.at[pl.ds(base, CH)], dv)

        @pl.when(c == 0)
        def _():
            pltpu.async_copy(hL.at[sv], rows, sem).wait()
            pltpu.sync_copy(rows, acc.at[dv], add=True)

        @pl.when(c == 1)
        def _():
            pltpu.async_copy(hR.at[sv], rows, sem).wait()
            pltpu.sync_copy(rows, acc.at[dv], add=True)

    plsc.subcore_barrier()

    @pl.when(c == 0)
    def _():
        pltpu.sync_copy(acc.at[pl.ds(s * ROWS_PER_TILE, ROWS_PER_TILE)],
                        aggL.at[pl.ds(s * ROWS_PER_TILE, ROWS_PER_TILE)])

    @pl.when(c == 1)
    def _():
        pltpu.sync_copy(acc.at[pl.ds(s * ROWS_PER_TILE, ROWS_PER_TILE)],
                        aggR.at[pl.ds(s * ROWS_PER_TILE, ROWS_PER_TILE)])


@functools.lru_cache(maxsize=None)
def _sc_segsum():
    return pl.kernel(
        _sc_segsum_body,
        out_type=(jax.ShapeDtypeStruct((N_PAD, H), jnp.float32),
                  jax.ShapeDtypeStruct((N_PAD, H), jnp.float32)),
        mesh=_mesh(),
        scratch_types=[
            pltpu.VMEM_SHARED((N_PAD, H), jnp.float32),
            pltpu.VMEM((ZROWS, H), jnp.float32),
            pltpu.VMEM((CH,), jnp.int32),
            pltpu.VMEM((CH,), jnp.int32),
            pltpu.VMEM((CH, H), jnp.float32),
            pltpu.SemaphoreType.DMA,
        ],
        compiler_params=pltpu.CompilerParams(use_tc_tiling_on_sc=False),
    )


def _sc_edge_body(P, Q, srcu, dstu, ef, sv, dv, bp, bq, sem):
    c = lax.axis_index("c")
    s = lax.axis_index("s")
    w = s * NC + c

    @pl.loop(0, EDGE_ITERS)
    def _it(i):
        cid = w + N_WORKERS * i

        @pl.when(cid < N_CHUNKS_EDGE)
        def _():
            base = cid * CH
            pltpu.sync_copy(srcu.at[pl.ds(base, CH)], sv)
            pltpu.sync_copy(dstu.at[pl.ds(base, CH)], dv)
            pltpu.async_copy(P.at[sv], bp, sem).wait()
            pltpu.async_copy(Q.at[dv], bq, sem).wait()

            @pl.loop(0, CH)
            def _row(r):
                for k in range(D // 16):
                    bp[r, k * 16:(k + 1) * 16] = (
                        bp[r, k * 16:(k + 1) * 16] + bq[r, k * 16:(k + 1) * 16])

            pltpu.sync_copy(bp, ef.at[pl.ds(base, CH)])


@functools.lru_cache(maxsize=None)
def _sc_edge():
    return pl.kernel(
        _sc_edge_body,
        out_type=jax.ShapeDtypeStruct((N_EDGES, D), jnp.float32),
        mesh=_mesh(),
        scratch_types=[
            pltpu.VMEM((CH,), jnp.int32),
            pltpu.VMEM((CH,), jnp.int32),
            pltpu.VMEM((CH, D), jnp.float32),
            pltpu.VMEM((CH, D), jnp.float32),
            pltpu.SemaphoreType.DMA,
        ],
        compiler_params=pltpu.CompilerParams(use_tc_tiling_on_sc=False),
    )


_BLK = 2000


def _tc_embed_body(c_ref, w_ref, b_ref, hL_ref, hR_ref):
    c = c_ref[...]
    w = w_ref[...]
    h = c[:, 0:1] * w[0:1, :] + c[:, 1:2] * w[1:2, :] + b_ref[...]
    hL_ref[...] = h[:, :H]
    hR_ref[...] = h[:, H:]


def _tc_embed(coords, W_node, b_node):
    g = N_NODES // _BLK
    return pl.pallas_call(
        _tc_embed_body,
        grid=(g,),
        in_specs=[
            pl.BlockSpec((_BLK, 2), lambda i: (i, 0)),
            pl.BlockSpec((2, D), lambda i: (0, 0)),
            pl.BlockSpec((1, D), lambda i: (0, 0)),
        ],
        out_specs=[
            pl.BlockSpec((_BLK, H), lambda i: (i, 0)),
            pl.BlockSpec((_BLK, H), lambda i: (i, 0)),
        ],
        out_shape=[jax.ShapeDtypeStruct((N_NODES, H), jnp.float32),
                   jax.ShapeDtypeStruct((N_NODES, H), jnp.float32)],
    )(coords, W_node, b_node.reshape(1, D))


def _tc_update_body(hL_ref, hR_ref, aL_ref, aR_ref, w_ref, b_ref,
                    oL_ref, oR_ref):
    agg = jnp.concatenate([aL_ref[...], aR_ref[...]], axis=1)
    z = jnp.dot(agg, w_ref[...], preferred_element_type=jnp.float32) + b_ref[...]
    z = jnp.where(z >= 0, z, 0.01 * z)
    oL_ref[...] = hL_ref[...] + z[:, :H]
    oR_ref[...] = hR_ref[...] + z[:, H:]


def _tc_update(hL, hR, aL, aR, W, b):
    g = N_NODES // _BLK
    return pl.pallas_call(
        _tc_update_body,
        grid=(g,),
        in_specs=[
            pl.BlockSpec((_BLK, H), lambda i: (i, 0)),
            pl.BlockSpec((_BLK, H), lambda i: (i, 0)),
            pl.BlockSpec((_BLK, H), lambda i: (i, 0)),
            pl.BlockSpec((_BLK, H), lambda i: (i, 0)),
            pl.BlockSpec((D, D), lambda i: (0, 0)),
            pl.BlockSpec((1, D), lambda i: (0, 0)),
        ],
        out_specs=[
            pl.BlockSpec((_BLK, H), lambda i: (i, 0)),
            pl.BlockSpec((_BLK, H), lambda i: (i, 0)),
        ],
        out_shape=[jax.ShapeDtypeStruct((N_NODES, H), jnp.float32),
                   jax.ShapeDtypeStruct((N_NODES, H), jnp.float32)],
    )(hL, hR, aL, aR, W, b.reshape(1, D))


def _tc_proj_body(hL_ref, hR_ref, w_ref, b_ref, p_ref, q_ref):
    h = jnp.concatenate([hL_ref[...], hR_ref[...]], axis=1)
    w = w_ref[...]
    p_ref[...] = jnp.dot(h, w[:D], preferred_element_type=jnp.float32) + b_ref[...]
    q_ref[...] = jnp.dot(h, w[D:], preferred_element_type=jnp.float32)


def _tc_proj(hL, hR, W_edge, b_edge):
    g = N_NODES // _BLK
    return pl.pallas_call(
        _tc_proj_body,
        grid=(g,),
        in_specs=[
            pl.BlockSpec((_BLK, H), lambda i: (i, 0)),
            pl.BlockSpec((_BLK, H), lambda i: (i, 0)),
            pl.BlockSpec((2 * D, D), lambda i: (0, 0)),
            pl.BlockSpec((1, D), lambda i: (0, 0)),
        ],
        out_specs=[
            pl.BlockSpec((_BLK, D), lambda i: (i, 0)),
            pl.BlockSpec((_BLK, D), lambda i: (i, 0)),
        ],
        out_shape=[jax.ShapeDtypeStruct((N_NODES, D), jnp.float32),
                   jax.ShapeDtypeStruct((N_NODES, D), jnp.float32)],
    )(hL, hR, W_edge, b_edge.reshape(1, D))


def kernel(coords, edge_index, W_node, b_node, W_g0, b_g0, W_g1, b_g1,
           W_g2, b_g2, W_edge, b_edge):
    src = edge_index[0]
    dst = edge_index[1]
    pad = E_PAD - N_EDGES
    srcp = jnp.concatenate([src, jnp.zeros((pad,), jnp.int32)])
    dstp = jnp.concatenate([dst, jnp.full((pad,), N_NODES, jnp.int32)])

    hL, hR = _tc_embed(coords, W_node, b_node)
    for W, b in ((W_g0, b_g0), (W_g1, b_g1), (W_g2, b_g2)):
        aL, aR = _sc_segsum()(hL, hR, srcp, dstp)
        hL, hR = _tc_update(hL, hR, aL[:N_NODES], aR[:N_NODES], W, b)
    P, Q = _tc_proj(hL, hR, W_edge, b_edge)
    return _sc_edge()(P, Q, src, dst)


# speedup vs baseline: 4.4343x; 1.4422x over previous
import functools

import jax
import jax.numpy as jnp
from jax import lax
from jax.experimental import pallas as pl
from jax.experimental.pallas import tpu as pltpu
from jax.experimental.pallas import tpu_sc as plsc

N_NODES = 50000
N_EDGES = 800000
D = 64
H = 32

NC = 2
NS = 16
CH = 128

N_PAD = 50176
ROWS_PER_TILE = N_PAD // NS
E_PAD = 819200
CHUNKS_PER_TILE = E_PAD // (NS * CH)

N_CHUNKS_EDGE = N_EDGES // CH
N_WORKERS = NC * NS
EDGE_ITERS = (N_CHUNKS_EDGE + N_WORKERS - 1) // N_WORKERS

@functools.lru_cache(maxsize=None)
def _mesh():
    return plsc.VectorSubcoreMesh(
        core_axis_name="c", subcore_axis_name="s",
        num_cores=NC, num_subcores=NS)


SB = 2
N_BATCH = CHUNKS_PER_TILE // SB
BATCH_E = SB * CH
ZCOPIES = ROWS_PER_TILE // BATCH_E
ZREM = ROWS_PER_TILE - ZCOPIES * BATCH_E


def _sc_segsum_body(hL, hR, sd3, aggL, aggR, acc, ivb, rows2, gsem, ssem):
    c = lax.axis_index("c")
    s = lax.axis_index("s")

    @pl.loop(0, BATCH_E)
    def _zero(r):
        rows2[0, r, 0:16] = jnp.zeros((16,), jnp.float32)
        rows2[0, r, 16:32] = jnp.zeros((16,), jnp.float32)

    base_r = s * ROWS_PER_TILE
    for j in range(ZCOPIES):
        pltpu.sync_copy(rows2.at[0],
                        acc.at[pl.ds(base_r + j * BATCH_E, BATCH_E)])
    pltpu.sync_copy(rows2.at[0, pl.ds(0, ZREM)],
                    acc.at[pl.ds(base_r + ZCOPIES * BATCH_E, ZREM)])
    plsc.subcore_barrier()

    @pl.loop(0, N_BATCH)
    def _batch(k):
        b = k % 2

        @pl.when(k >= 2)
        def _():
            pltpu.make_async_copy(
                hL.at[pl.ds(0, BATCH_E)], rows2.at[b], ssem.at[b]).wait()

        row0 = s * CHUNKS_PER_TILE + k * SB
        pltpu.sync_copy(sd3.at[pl.ds(row0, SB)], ivb.at[b])

        @pl.when(c == 0)
        def _():
            for j in range(SB):
                pltpu.async_copy(hL.at[ivb.at[b, j, 0]],
                                 rows2.at[b, pl.ds(j * CH, CH)], gsem)

        @pl.when(c == 1)
        def _():
            for j in range(SB):
                pltpu.async_copy(hR.at[ivb.at[b, j, 0]],
                                 rows2.at[b, pl.ds(j * CH, CH)], gsem)

        pltpu.make_async_copy(
            hL.at[pl.ds(0, BATCH_E)], rows2.at[b], gsem).wait()

        for j in range(SB):
            pltpu.async_copy(rows2.at[b, pl.ds(j * CH, CH)],
                             acc.at[ivb.at[b, j, 1]], ssem.at[b], add=True)

    for b in range(2):
        pltpu.make_async_copy(
            hL.at[pl.ds(0, BATCH_E)], rows2.at[b], ssem.at[b]).wait()

    plsc.subcore_barrier()

    @pl.when(c == 0)
    def _():
        pltpu.sync_copy(acc.at[pl.ds(s * ROWS_PER_TILE, ROWS_PER_TILE)],
                        aggL.at[pl.ds(s * ROWS_PER_TILE, ROWS_PER_TILE)])

    @pl.when(c == 1)
    def _():
        pltpu.sync_copy(acc.at[pl.ds(s * ROWS_PER_TILE, ROWS_PER_TILE)],
                        aggR.at[pl.ds(s * ROWS_PER_TILE, ROWS_PER_TILE)])


@functools.lru_cache(maxsize=None)
def _sc_segsum():
    return pl.kernel(
        _sc_segsum_body,
        out_type=(jax.ShapeDtypeStruct((N_PAD, H), jnp.float32),
                  jax.ShapeDtypeStruct((N_PAD, H), jnp.float32)),
        mesh=_mesh(),
        scratch_types=[
            pltpu.VMEM_SHARED((N_PAD, H), jnp.float32),
            pltpu.VMEM((2, SB, 2, CH), jnp.int32),
            pltpu.VMEM((2, BATCH_E, H), jnp.float32),
            pltpu.SemaphoreType.DMA,
            pltpu.SemaphoreType.DMA((2,)),
        ],
        compiler_params=pltpu.CompilerParams(use_tc_tiling_on_sc=False),
    )


def _sc_edge_body(P, Q, srcu2, dstu2, ef, svb, dvb, bp2, bq2, gsem, osem):
    c = lax.axis_index("c")
    s = lax.axis_index("s")
    w = s * NC + c
    n = jnp.where(w < N_CHUNKS_EDGE - (EDGE_ITERS - 1) * N_WORKERS,
                  EDGE_ITERS, EDGE_ITERS - 1)

    def _fire(i, b):
        cid = w + N_WORKERS * i
        pltpu.sync_copy(srcu2.at[pl.ds(cid, 1)], svb.at[b])
        pltpu.sync_copy(dstu2.at[pl.ds(cid, 1)], dvb.at[b])
        pltpu.async_copy(P.at[svb.at[b, 0]], bp2.at[b], gsem.at[b])
        pltpu.async_copy(Q.at[dvb.at[b, 0]], bq2.at[b], gsem.at[b])

    _fire(0, 0)

    @pl.loop(0, n)
    def _it(i):
        b = i % 2
        q = 1 - b
        pltpu.make_async_copy(P.at[pl.ds(0, CH)], bp2.at[b], gsem.at[b]).wait()
        pltpu.make_async_copy(P.at[pl.ds(0, CH)], bq2.at[b], gsem.at[b]).wait()

        @pl.when(i + 1 < n)
        def _():
            @pl.when(i >= 1)
            def _():
                pltpu.make_async_copy(
                    bp2.at[q], ef.at[pl.ds(0, CH)], osem.at[q]).wait()
            _fire(i + 1, q)

        @pl.loop(0, CH, unroll=4)
        def _row(r):
            for k in range(D // 16):
                bp2[b, r, k * 16:(k + 1) * 16] = (
                    bp2[b, r, k * 16:(k + 1) * 16]
                    + bq2[b, r, k * 16:(k + 1) * 16])

        pltpu.async_copy(bp2.at[b], ef.at[pl.ds((w + N_WORKERS * i) * CH, CH)],
                         osem.at[b])

    for b in range(2):
        pltpu.make_async_copy(bp2.at[b], ef.at[pl.ds(0, CH)], osem.at[b]).wait()


@functools.lru_cache(maxsize=None)
def _sc_edge():
    return pl.kernel(
        _sc_edge_body,
        out_type=jax.ShapeDtypeStruct((N_EDGES, D), jnp.float32),
        mesh=_mesh(),
        scratch_types=[
            pltpu.VMEM((2, 1, CH), jnp.int32),
            pltpu.VMEM((2, 1, CH), jnp.int32),
            pltpu.VMEM((2, CH, D), jnp.float32),
            pltpu.VMEM((2, CH, D), jnp.float32),
            pltpu.SemaphoreType.DMA((2,)),
            pltpu.SemaphoreType.DMA((2,)),
        ],
        compiler_params=pltpu.CompilerParams(use_tc_tiling_on_sc=False),
    )


_BLK = 2000


def _tc_embed_body(c_ref, w_ref, b_ref, hL_ref, hR_ref):
    c = c_ref[...]
    w = w_ref[...]
    h = c[:, 0:1] * w[0:1, :] + c[:, 1:2] * w[1:2, :] + b_ref[...]
    hL_ref[...] = h[:, :H]
    hR_ref[...] = h[:, H:]


def _tc_embed(coords, W_node, b_node):
    g = N_NODES // _BLK
    return pl.pallas_call(
        _tc_embed_body,
        grid=(g,),
        in_specs=[
            pl.BlockSpec((_BLK, 2), lambda i: (i, 0)),
            pl.BlockSpec((2, D), lambda i: (0, 0)),
            pl.BlockSpec((1, D), lambda i: (0, 0)),
        ],
        out_specs=[
            pl.BlockSpec((_BLK, H), lambda i: (i, 0)),
            pl.BlockSpec((_BLK, H), lambda i: (i, 0)),
        ],
        out_shape=[jax.ShapeDtypeStruct((N_NODES, H), jnp.float32),
                   jax.ShapeDtypeStruct((N_NODES, H), jnp.float32)],
    )(coords, W_node, b_node.reshape(1, D))


def _tc_update_body(hL_ref, hR_ref, aL_ref, aR_ref, w_ref, b_ref,
                    oL_ref, oR_ref):
    agg = jnp.concatenate([aL_ref[...], aR_ref[...]], axis=1)
    z = jnp.dot(agg, w_ref[...], preferred_element_type=jnp.float32) + b_ref[...]
    z = jnp.where(z >= 0, z, 0.01 * z)
    oL_ref[...] = hL_ref[...] + z[:, :H]
    oR_ref[...] = hR_ref[...] + z[:, H:]


def _tc_update(hL, hR, aL, aR, W, b):
    g = N_NODES // _BLK
    return pl.pallas_call(
        _tc_update_body,
        grid=(g,),
        in_specs=[
            pl.BlockSpec((_BLK, H), lambda i: (i, 0)),
            pl.BlockSpec((_BLK, H), lambda i: (i, 0)),
            pl.BlockSpec((_BLK, H), lambda i: (i, 0)),
            pl.BlockSpec((_BLK, H), lambda i: (i, 0)),
            pl.BlockSpec((D, D), lambda i: (0, 0)),
            pl.BlockSpec((1, D), lambda i: (0, 0)),
        ],
        out_specs=[
            pl.BlockSpec((_BLK, H), lambda i: (i, 0)),
            pl.BlockSpec((_BLK, H), lambda i: (i, 0)),
        ],
        out_shape=[jax.ShapeDtypeStruct((N_NODES, H), jnp.float32),
                   jax.ShapeDtypeStruct((N_NODES, H), jnp.float32)],
    )(hL, hR, aL, aR, W, b.reshape(1, D))


def _tc_proj_body(hL_ref, hR_ref, w_ref, b_ref, p_ref, q_ref):
    h = jnp.concatenate([hL_ref[...], hR_ref[...]], axis=1)
    w = w_ref[...]
    p_ref[...] = jnp.dot(h, w[:D], preferred_element_type=jnp.float32) + b_ref[...]
    q_ref[...] = jnp.dot(h, w[D:], preferred_element_type=jnp.float32)


def _tc_proj(hL, hR, W_edge, b_edge):
    g = N_NODES // _BLK
    return pl.pallas_call(
        _tc_proj_body,
        grid=(g,),
        in_specs=[
            pl.BlockSpec((_BLK, H), lambda i: (i, 0)),
            pl.BlockSpec((_BLK, H), lambda i: (i, 0)),
            pl.BlockSpec((2 * D, D), lambda i: (0, 0)),
            pl.BlockSpec((1, D), lambda i: (0, 0)),
        ],
        out_specs=[
            pl.BlockSpec((_BLK, D), lambda i: (i, 0)),
            pl.BlockSpec((_BLK, D), lambda i: (i, 0)),
        ],
        out_shape=[jax.ShapeDtypeStruct((N_NODES, D), jnp.float32),
                   jax.ShapeDtypeStruct((N_NODES, D), jnp.float32)],
    )(hL, hR, W_edge, b_edge.reshape(1, D))


def kernel(coords, edge_index, W_node, b_node, W_g0, b_g0, W_g1, b_g1,
           W_g2, b_g2, W_edge, b_edge):
    src = edge_index[0]
    dst = edge_index[1]
    pad = E_PAD - N_EDGES
    srcp2 = jnp.concatenate([src, jnp.zeros((pad,), jnp.int32)]).reshape(-1, CH)
    dstp2 = jnp.concatenate([dst, jnp.full((pad,), N_NODES, jnp.int32)]
                            ).reshape(-1, CH)
    sd3 = jnp.stack([srcp2, dstp2], axis=1)
    srcu2 = src.reshape(-1, CH)
    dstu2 = dst.reshape(-1, CH)

    hL, hR = _tc_embed(coords, W_node, b_node)
    for W, b in ((W_g0, b_g0), (W_g1, b_g1), (W_g2, b_g2)):
        aL, aR = _sc_segsum()(hL, hR, sd3)
        hL, hR = _tc_update(hL, hR, aL, aR, W, b)
    P, Q = _tc_proj(hL, hR, W_edge, b_edge)
    return _sc_edge()(P, Q, srcu2, dstu2)


# speedup vs baseline: 5.4767x; 1.2351x over previous
import functools

import jax
import jax.numpy as jnp
from jax import lax
from jax.experimental import pallas as pl
from jax.experimental.pallas import tpu as pltpu
from jax.experimental.pallas import tpu_sc as plsc

N_NODES = 50000
N_EDGES = 800000
D = 64
H = 32

NC = 2
NS = 16
CH = 128

N_PAD = 50176
ROWS_PER_TILE = N_PAD // NS
E_PAD = 819200
CHUNKS_PER_TILE = E_PAD // (NS * CH)

N_CHUNKS_EDGE = N_EDGES // CH
N_WORKERS = NC * NS
EDGE_ITERS = (N_CHUNKS_EDGE + N_WORKERS - 1) // N_WORKERS

@functools.lru_cache(maxsize=None)
def _mesh():
    return plsc.VectorSubcoreMesh(
        core_axis_name="c", subcore_axis_name="s",
        num_cores=NC, num_subcores=NS)


NBUF = 4
IBLK = 16
N_IBLK = CHUNKS_PER_TILE // IBLK
ZR = 256
ZCOPIES = ROWS_PER_TILE // ZR
ZREM = ROWS_PER_TILE - ZCOPIES * ZR


def _sc_segsum_body(hL, hR, sd3, aggL, aggR, acc, ivb, rows, gsem, ssem, isem):
    c = lax.axis_index("c")
    s = lax.axis_index("s")
    tile_row0 = s * CHUNKS_PER_TILE

    pltpu.async_copy(sd3.at[pl.ds(tile_row0, IBLK)], ivb.at[0], isem.at[0])

    @pl.loop(0, ZR)
    def _zero(r):
        rows[0, r, 0:16] = jnp.zeros((16,), jnp.float32)
        rows[0, r, 16:32] = jnp.zeros((16,), jnp.float32)

    base_r = s * ROWS_PER_TILE
    for j in range(ZCOPIES):
        pltpu.sync_copy(rows.at[0, pl.ds(0, ZR)],
                        acc.at[pl.ds(base_r + j * ZR, ZR)])
    pltpu.sync_copy(rows.at[0, pl.ds(0, ZREM)],
                    acc.at[pl.ds(base_r + ZCOPIES * ZR, ZREM)])
    plsc.subcore_barrier()

    def _gather(i):
        blk2 = (i // IBLK) % 2
        off = i % IBLK
        j = i % NBUF

        @pl.when(c == 0)
        def _():
            pltpu.async_copy(hL.at[ivb.at[blk2, off, 0]], rows.at[j],
                             gsem.at[j])

        @pl.when(c == 1)
        def _():
            pltpu.async_copy(hR.at[ivb.at[blk2, off, 0]], rows.at[j],
                             gsem.at[j])

    def _scatter(i):
        blk2 = (i // IBLK) % 2
        off = i % IBLK
        j = i % NBUF
        pltpu.make_async_copy(hL.at[pl.ds(0, CH)], rows.at[j],
                              gsem.at[j]).wait()
        pltpu.async_copy(rows.at[j], acc.at[ivb.at[blk2, off, 1]],
                         ssem.at[j], add=True)

    @pl.loop(0, CHUNKS_PER_TILE)
    def _chunk(i):
        @pl.when(i % IBLK == 0)
        def _():
            blk = i // IBLK
            pltpu.make_async_copy(sd3.at[pl.ds(0, IBLK)], ivb.at[blk % 2],
                                  isem.at[blk % 2]).wait()

        @pl.when(i % IBLK == NBUF)
        def _():
            blk = i // IBLK

            @pl.when(blk + 1 < N_IBLK)
            def _():
                pltpu.async_copy(
                    sd3.at[pl.ds(tile_row0 + (blk + 1) * IBLK, IBLK)],
                    ivb.at[(blk + 1) % 2], isem.at[(blk + 1) % 2])

        @pl.when(i >= NBUF)
        def _():
            j = i % NBUF
            pltpu.make_async_copy(rows.at[j], acc.at[pl.ds(0, CH)],
                                  ssem.at[j]).wait()

        _gather(i)

        @pl.when(i >= 2)
        def _():
            _scatter(i - 2)

    _scatter(CHUNKS_PER_TILE - 2)
    _scatter(CHUNKS_PER_TILE - 1)
    for j in range(NBUF):
        pltpu.make_async_copy(rows.at[j], acc.at[pl.ds(0, CH)],
                              ssem.at[j]).wait()

    plsc.subcore_barrier()

    @pl.when(c == 0)
    def _():
        pltpu.sync_copy(acc.at[pl.ds(s * ROWS_PER_TILE, ROWS_PER_TILE)],
                        aggL.at[pl.ds(s * ROWS_PER_TILE, ROWS_PER_TILE)])

    @pl.when(c == 1)
    def _():
        pltpu.sync_copy(acc.at[pl.ds(s * ROWS_PER_TILE, ROWS_PER_TILE)],
                        aggR.at[pl.ds(s * ROWS_PER_TILE, ROWS_PER_TILE)])


@functools.lru_cache(maxsize=None)
def _sc_segsum():
    return pl.kernel(
        _sc_segsum_body,
        out_type=(jax.ShapeDtypeStruct((N_PAD, H), jnp.float32),
                  jax.ShapeDtypeStruct((N_PAD, H), jnp.float32)),
        mesh=_mesh(),
        scratch_types=[
            pltpu.VMEM_SHARED((N_PAD, H), jnp.float32),
            pltpu.VMEM((2, IBLK, 2, CH), jnp.int32),
            pltpu.VMEM((NBUF, CH, H), jnp.float32),
            pltpu.SemaphoreType.DMA((NBUF,)),
            pltpu.SemaphoreType.DMA((NBUF,)),
            pltpu.SemaphoreType.DMA((2,)),
        ],
        compiler_params=pltpu.CompilerParams(use_tc_tiling_on_sc=False),
    )


EDGE_BASE = N_CHUNKS_EDGE // N_WORKERS
EDGE_XTRA = N_CHUNKS_EDGE - EDGE_BASE * N_WORKERS


def _sc_edge_body(P, Q, sdu3, ef, ivb, bp2, bq2, gsem, osem):
    c = lax.axis_index("c")
    s = lax.axis_index("s")
    w = s * NC + c
    start = w * EDGE_BASE + jnp.minimum(w, EDGE_XTRA)
    n = EDGE_BASE + jnp.where(w < EDGE_XTRA, 1, 0)

    pltpu.sync_copy(sdu3.at[pl.ds(start, EDGE_ITERS)], ivb)

    def _fire(i):
        b = i % 2
        pltpu.async_copy(P.at[ivb.at[i, 0]], bp2.at[b], gsem.at[b])
        pltpu.async_copy(Q.at[ivb.at[i, 1]], bq2.at[b], gsem.at[b])

    def _finish(i):
        b = i % 2
        pltpu.make_async_copy(P.at[pl.ds(0, CH)], bp2.at[b], gsem.at[b]).wait()
        pltpu.make_async_copy(P.at[pl.ds(0, CH)], bq2.at[b], gsem.at[b]).wait()

        @pl.loop(0, CH, unroll=8)
        def _row(r):
            for k in range(D // 16):
                bp2[b, r, k * 16:(k + 1) * 16] = (
                    bp2[b, r, k * 16:(k + 1) * 16]
                    + bq2[b, r, k * 16:(k + 1) * 16])

        pltpu.async_copy(bp2.at[b], ef.at[pl.ds((start + i) * CH, CH)],
                         osem.at[b])

    _fire(0)

    @pl.loop(1, n)
    def _it(i):
        b = i % 2

        @pl.when(i >= 2)
        def _():
            pltpu.make_async_copy(bp2.at[b], ef.at[pl.ds(0, CH)],
                                  osem.at[b]).wait()

        _fire(i)
        _finish(i - 1)

    _finish(n - 1)
    for b in range(2):
        pltpu.make_async_copy(bp2.at[b], ef.at[pl.ds(0, CH)], osem.at[b]).wait()


@functools.lru_cache(maxsize=None)
def _sc_edge():
    return pl.kernel(
        _sc_edge_body,
        out_type=jax.ShapeDtypeStruct((N_EDGES, D), jnp.float32),
        mesh=_mesh(),
        scratch_types=[
            pltpu.VMEM((EDGE_ITERS, 2, CH), jnp.int32),
            pltpu.VMEM((2, CH, D), jnp.float32),
            pltpu.VMEM((2, CH, D), jnp.float32),
            pltpu.SemaphoreType.DMA((2,)),
            pltpu.SemaphoreType.DMA((2,)),
        ],
        compiler_params=pltpu.CompilerParams(use_tc_tiling_on_sc=False),
    )


_BLK = 2000


def _tc_embed_body(c_ref, w_ref, b_ref, hL_ref, hR_ref):
    c = c_ref[...]
    w = w_ref[...]
    h = c[:, 0:1] * w[0:1, :] + c[:, 1:2] * w[1:2, :] + b_ref[...]
    hL_ref[...] = h[:, :H]
    hR_ref[...] = h[:, H:]


def _tc_embed(coords, W_node, b_node):
    g = N_NODES // _BLK
    return pl.pallas_call(
        _tc_embed_body,
        grid=(g,),
        in_specs=[
            pl.BlockSpec((_BLK, 2), lambda i: (i, 0)),
            pl.BlockSpec((2, D), lambda i: (0, 0)),
            pl.BlockSpec((1, D), lambda i: (0, 0)),
        ],
        out_specs=[
            pl.BlockSpec((_BLK, H), lambda i: (i, 0)),
            pl.BlockSpec((_BLK, H), lambda i: (i, 0)),
        ],
        out_shape=[jax.ShapeDtypeStruct((N_NODES, H), jnp.float32),
                   jax.ShapeDtypeStruct((N_NODES, H), jnp.float32)],
    )(coords, W_node, b_node.reshape(1, D))


def _tc_update_body(hL_ref, hR_ref, aL_ref, aR_ref, w_ref, b_ref,
                    oL_ref, oR_ref):
    agg = jnp.concatenate([aL_ref[...], aR_ref[...]], axis=1)
    z = jnp.dot(agg, w_ref[...], preferred_element_type=jnp.float32) + b_ref[...]
    z = jnp.where(z >= 0, z, 0.01 * z)
    oL_ref[...] = hL_ref[...] + z[:, :H]
    oR_ref[...] = hR_ref[...] + z[:, H:]


def _tc_update(hL, hR, aL, aR, W, b):
    g = N_NODES // _BLK
    return pl.pallas_call(
        _tc_update_body,
        grid=(g,),
        in_specs=[
            pl.BlockSpec((_BLK, H), lambda i: (i, 0)),
            pl.BlockSpec((_BLK, H), lambda i: (i, 0)),
            pl.BlockSpec((_BLK, H), lambda i: (i, 0)),
            pl.BlockSpec((_BLK, H), lambda i: (i, 0)),
            pl.BlockSpec((D, D), lambda i: (0, 0)),
            pl.BlockSpec((1, D), lambda i: (0, 0)),
        ],
        out_specs=[
            pl.BlockSpec((_BLK, H), lambda i: (i, 0)),
            pl.BlockSpec((_BLK, H), lambda i: (i, 0)),
        ],
        out_shape=[jax.ShapeDtypeStruct((N_NODES, H), jnp.float32),
                   jax.ShapeDtypeStruct((N_NODES, H), jnp.float32)],
    )(hL, hR, aL, aR, W, b.reshape(1, D))


def _tc_proj_body(hL_ref, hR_ref, w_ref, b_ref, p_ref, q_ref):
    h = jnp.concatenate([hL_ref[...], hR_ref[...]], axis=1)
    w = w_ref[...]
    p_ref[...] = jnp.dot(h, w[:D], preferred_element_type=jnp.float32) + b_ref[...]
    q_ref[...] = jnp.dot(h, w[D:], preferred_element_type=jnp.float32)


def _tc_proj(hL, hR, W_edge, b_edge):
    g = N_NODES // _BLK
    return pl.pallas_call(
        _tc_proj_body,
        grid=(g,),
        in_specs=[
            pl.BlockSpec((_BLK, H), lambda i: (i, 0)),
            pl.BlockSpec((_BLK, H), lambda i: (i, 0)),
            pl.BlockSpec((2 * D, D), lambda i: (0, 0)),
            pl.BlockSpec((1, D), lambda i: (0, 0)),
        ],
        out_specs=[
            pl.BlockSpec((_BLK, D), lambda i: (i, 0)),
            pl.BlockSpec((_BLK, D), lambda i: (i, 0)),
        ],
        out_shape=[jax.ShapeDtypeStruct((N_NODES, D), jnp.float32),
                   jax.ShapeDtypeStruct((N_NODES, D), jnp.float32)],
    )(hL, hR, W_edge, b_edge.reshape(1, D))


def kernel(coords, edge_index, W_node, b_node, W_g0, b_g0, W_g1, b_g1,
           W_g2, b_g2, W_edge, b_edge):
    src = edge_index[0]
    dst = edge_index[1]
    pad = E_PAD - N_EDGES
    srcp2 = jnp.concatenate([src, jnp.zeros((pad,), jnp.int32)]).reshape(-1, CH)
    dstp2 = jnp.concatenate([dst, jnp.full((pad,), N_NODES, jnp.int32)]
                            ).reshape(-1, CH)
    sd3 = jnp.stack([srcp2, dstp2], axis=1)
    sdu3 = jnp.concatenate([
        jnp.stack([src.reshape(-1, CH), dst.reshape(-1, CH)], axis=1),
        jnp.zeros((1, 2, CH), jnp.int32)])

    hL, hR = _tc_embed(coords, W_node, b_node)
    for W, b in ((W_g0, b_g0), (W_g1, b_g1), (W_g2, b_g2)):
        aL, aR = _sc_segsum()(hL, hR, sd3)
        hL, hR = _tc_update(hL, hR, aL, aR, W, b)
    P, Q = _tc_proj(hL, hR, W_edge, b_edge)
    return _sc_edge()(P, Q, sdu3)
